# Initial kernel scaffold; baseline (speedup 1.0000x reference)
import jax
import jax.numpy as jnp
from jax.experimental import pallas as pl


def kernel(x, edge_index, conv_W1, conv_b1, gn_w1, gn_b1, gn_ms1, conv_W2, conv_b2, gn_w2, gn_b2, gn_ms2, conv_W3, conv_b3, gn_w3, gn_b3, gn_ms3, lin1_W, lin1_b, lin2_W, lin2_b):
    raise NotImplementedError("write your pallas kernel here")


# speedup vs baseline: 10.0498x; 10.0498x over previous
import functools

import jax
import jax.numpy as jnp
from jax import lax
from jax.experimental import pallas as pl
from jax.experimental.pallas import tpu as pltpu
from jax.experimental.pallas import tpu_sc as plsc

N = 10000
E = 320000
D = 128
H = 128
FOUT = 768
EPS = 1e-5

NC = 2
NS = 16
NW = NC * NS
EPT = E // NW
EB = 80
NP = 10240
RPT = NP // NS
ZR = 128
DEGW = 16

_MESH = plsc.VectorSubcoreMesh(core_axis_name="c", subcore_axis_name="s")

RB = 1000
NB = N // RB


def _sc_degree(dst):
    @functools.partial(
        pl.kernel,
        out_type=jax.ShapeDtypeStruct((NC, NP, DEGW), jnp.float32),
        mesh=_MESH,
        scratch_types=[
            pltpu.VMEM((EB,), jnp.int32),
            pltpu.VMEM((EB, DEGW), jnp.float32),
            pltpu.VMEM((ZR, DEGW), jnp.float32),
            pltpu.VMEM_SHARED((NP, DEGW), jnp.float32),
        ],
    )
    def k(dst_hbm, out_hbm, dst_v, ones_v, zbuf, acc):
        cid = lax.axis_index("c")
        sid = lax.axis_index("s")
        wid = sid * NC + cid

        def fill(i, _):
            z16 = jnp.zeros((16,), jnp.float32)
            ones_row = jnp.ones((16,), jnp.float32)
            zbuf[i, :] = z16

            @pl.when(i < EB)
            def _():
                ones_v[i, :] = ones_row
            return 0

        lax.fori_loop(0, ZR, fill, 0)
        for r in range(RPT // ZR):
            pltpu.sync_copy(zbuf, acc.at[pl.ds(sid * RPT + r * ZR, ZR)])
        plsc.subcore_barrier()

        def body(j, _):
            base = wid * EPT + j * EB
            pltpu.sync_copy(dst_hbm.at[pl.ds(base, EB)], dst_v)
            pltpu.sync_copy(ones_v, acc.at[dst_v], add=True)
            return 0

        lax.fori_loop(0, EPT // EB, body, 0)
        plsc.subcore_barrier()
        pltpu.sync_copy(acc.at[pl.ds(sid * RPT, RPT)],
                        out_hbm.at[cid, pl.ds(sid * RPT, RPT)])

    return k(dst)


def _sc_aggregate(hp, src, dst):
    @functools.partial(
        pl.kernel,
        out_type=jax.ShapeDtypeStruct((NC, NP, H), jnp.float32),
        mesh=_MESH,
        scratch_types=[
            pltpu.VMEM((EB,), jnp.int32),
            pltpu.VMEM((EB,), jnp.int32),
            pltpu.VMEM((EB, H), jnp.float32),
            pltpu.VMEM((ZR, H), jnp.float32),
            pltpu.VMEM_SHARED((NP, H), jnp.float32),
            pltpu.SemaphoreType.DMA,
        ],
    )
    def k(hp_hbm, src_hbm, dst_hbm, out_hbm, src_v, dst_v, gbuf, zbuf, acc, sem):
        cid = lax.axis_index("c")
        sid = lax.axis_index("s")
        wid = sid * NC + cid

        def fill(i, _):
            z16 = jnp.zeros((16,), jnp.float32)
            for f in range(H // 16):
                zbuf[i, pl.ds(f * 16, 16)] = z16
            return 0

        lax.fori_loop(0, ZR, fill, 0)
        for r in range(RPT // ZR):
            pltpu.sync_copy(zbuf, acc.at[pl.ds(sid * RPT + r * ZR, ZR)])
        plsc.subcore_barrier()

        def body(j, _):
            base = wid * EPT + j * EB
            pltpu.sync_copy(src_hbm.at[pl.ds(base, EB)], src_v)
            pltpu.sync_copy(dst_hbm.at[pl.ds(base, EB)], dst_v)
            pltpu.async_copy(hp_hbm.at[src_v], gbuf, sem).wait()
            pltpu.sync_copy(gbuf, acc.at[dst_v], add=True)
            return 0

        lax.fori_loop(0, EPT // EB, body, 0)
        plsc.subcore_barrier()
        pltpu.sync_copy(acc.at[pl.ds(sid * RPT, RPT)],
                        out_hbm.at[cid, pl.ds(sid * RPT, RPT)])

    return k(hp, src, dst)


def _dinv_block(deg_ref):
    deg = deg_ref[0, :, 0:1] + deg_ref[1, :, 0:1] + 1.0
    return lax.rsqrt(deg)


_DEG_SPEC = pl.BlockSpec((NC, RB, DEGW), lambda i: (0, i, 0))
_ROW_SPEC = pl.BlockSpec((RB, H), lambda i: (i, 0))
_FULL_W = pl.BlockSpec((H, H), lambda i: (0, 0))
_VEC_SPEC = pl.BlockSpec((1, H), lambda i: (0, 0))


def _mm(a, b):
    return jnp.dot(a, b, preferred_element_type=jnp.float32,
                   precision=lax.Precision.HIGHEST)


def _tc_first(deg, x, W1):
    def body(deg_ref, x_ref, w_ref, hp_ref):
        hp_ref[...] = _mm(x_ref[...], w_ref[...]) * _dinv_block(deg_ref)

    return pl.pallas_call(
        body,
        grid=(NB,),
        in_specs=[_DEG_SPEC, _ROW_SPEC, _FULL_W],
        out_specs=_ROW_SPEC,
        out_shape=jax.ShapeDtypeStruct((N, H), jnp.float32),
    )(deg, x, W1)


def _tc_stats(S, hp, deg, b, gw, gb, gms):
    s_spec = pl.BlockSpec((NC, RB, H), lambda i: (0, i, 0))

    def body(s_ref, hp_ref, deg_ref, b_ref, gw_ref, gb_ref, gms_ref,
             out_ref, a_ref, c_ref, sum1, sum2):
        i = pl.program_id(0)

        @pl.when(i == 0)
        def _():
            sum1[...] = jnp.zeros_like(sum1)
            sum2[...] = jnp.zeros_like(sum2)

        dinv = _dinv_block(deg_ref)
        o = dinv * (s_ref[0] + s_ref[1] + hp_ref[...]) + b_ref[...]
        out_ref[...] = o
        sum1[...] += jnp.sum(o, axis=0, keepdims=True)
        sum2[...] += jnp.sum(o * o, axis=0, keepdims=True)

        @pl.when(i == NB - 1)
        def _():
            mean = sum1[...] * (1.0 / N)
            ex2 = sum2[...] * (1.0 / N)
            ms = gms_ref[...]
            var = ex2 - (2.0 * ms - ms * ms) * mean * mean
            a = gw_ref[...] / jnp.sqrt(var + EPS)
            a_ref[...] = a
            c_ref[...] = gb_ref[...] - a * ms * mean

    return pl.pallas_call(
        body,
        grid=(NB,),
        in_specs=[s_spec, _ROW_SPEC, _DEG_SPEC, _VEC_SPEC, _VEC_SPEC,
                  _VEC_SPEC, _VEC_SPEC],
        out_specs=[_ROW_SPEC, _VEC_SPEC, _VEC_SPEC],
        out_shape=[
            jax.ShapeDtypeStruct((N, H), jnp.float32),
            jax.ShapeDtypeStruct((1, H), jnp.float32),
            jax.ShapeDtypeStruct((1, H), jnp.float32),
        ],
        scratch_shapes=[pltpu.VMEM((1, H), jnp.float32),
                        pltpu.VMEM((1, H), jnp.float32)],
    )(S, hp, deg, b, gw, gb, gms)


def _tc_ad1(out1, a1, c1, deg, W2):
    def body(o_ref, a_ref, c_ref, deg_ref, w_ref, y_ref, hp_ref):
        y = jnp.maximum(a_ref[...] * o_ref[...] + c_ref[...], 0.0)
        y_ref[...] = y
        hp_ref[...] = _mm(y, w_ref[...]) * _dinv_block(deg_ref)

    return pl.pallas_call(
        body,
        grid=(NB,),
        in_specs=[_ROW_SPEC, _VEC_SPEC, _VEC_SPEC, _DEG_SPEC, _FULL_W],
        out_specs=[_ROW_SPEC, _ROW_SPEC],
        out_shape=[jax.ShapeDtypeStruct((N, H), jnp.float32),
                   jax.ShapeDtypeStruct((N, H), jnp.float32)],
    )(out1, a1, c1, deg, W2)


def _tc_ad2(out2, a2, c2, y1, deg, W3):
    def body(o_ref, a_ref, c_ref, y1_ref, deg_ref, w_ref, zmax_ref, hp_ref):
        y1 = y1_ref[...]
        y2 = jnp.maximum(a_ref[...] * o_ref[...] + c_ref[...], 0.0)
        zmax_ref[...] = jnp.maximum(y1, y2)
        hp_ref[...] = _mm(y2 + y1, w_ref[...]) * _dinv_block(deg_ref)

    return pl.pallas_call(
        body,
        grid=(NB,),
        in_specs=[_ROW_SPEC, _VEC_SPEC, _VEC_SPEC, _ROW_SPEC, _DEG_SPEC,
                  _FULL_W],
        out_specs=[_ROW_SPEC, _ROW_SPEC],
        out_shape=[jax.ShapeDtypeStruct((N, H), jnp.float32),
                   jax.ShapeDtypeStruct((N, H), jnp.float32)],
    )(out2, a2, c2, y1, deg, W3)


def _tc_final(out3, a3, c3, zmax, lin1_W, lin1_b, lin2_W, lin2_b):
    def body(o_ref, a_ref, c_ref, zm_ref, w1_ref, b1_ref, w2_ref, b2_ref,
             out_ref):
        y3 = jnp.maximum(a_ref[...] * o_ref[...] + c_ref[...], 0.0)
        z = jnp.maximum(zm_ref[...], y3)
        t = jnp.maximum(_mm(z, w1_ref[...]) + b1_ref[...], 0.0)
        out_ref[...] = _mm(t, w2_ref[...]) + b2_ref[...]

    return pl.pallas_call(
        body,
        grid=(NB,),
        in_specs=[
            _ROW_SPEC, _VEC_SPEC, _VEC_SPEC, _ROW_SPEC, _FULL_W, _VEC_SPEC,
            pl.BlockSpec((H, FOUT), lambda i: (0, 0)),
            pl.BlockSpec((1, FOUT), lambda i: (0, 0)),
        ],
        out_specs=pl.BlockSpec((RB, FOUT), lambda i: (i, 0)),
        out_shape=jax.ShapeDtypeStruct((N, FOUT), jnp.float32),
    )(out3, a3, c3, zmax, lin1_W, lin1_b, lin2_W, lin2_b)


def kernel(x, edge_index, conv_W1, conv_b1, gn_w1, gn_b1, gn_ms1,
           conv_W2, conv_b2, gn_w2, gn_b2, gn_ms2,
           conv_W3, conv_b3, gn_w3, gn_b3, gn_ms3,
           lin1_W, lin1_b, lin2_W, lin2_b):
    src = edge_index[0]
    dst = edge_index[1]
    r = lambda v: v.reshape(1, -1)

    deg = _sc_degree(dst)
    hp1 = _tc_first(deg, x, conv_W1)
    S1 = _sc_aggregate(hp1, src, dst)
    out1, a1, c1 = _tc_stats(S1, hp1, deg, r(conv_b1), r(gn_w1), r(gn_b1),
                             r(gn_ms1))
    y1, hp2 = _tc_ad1(out1, a1, c1, deg, conv_W2)
    S2 = _sc_aggregate(hp2, src, dst)
    out2, a2, c2 = _tc_stats(S2, hp2, deg, r(conv_b2), r(gn_w2), r(gn_b2),
                             r(gn_ms2))
    zmax2, hp3 = _tc_ad2(out2, a2, c2, y1, deg, conv_W3)
    S3 = _sc_aggregate(hp3, src, dst)
    out3, a3, c3 = _tc_stats(S3, hp3, deg, r(conv_b3), r(gn_w3), r(gn_b3),
                             r(gn_ms3))
    o = _tc_final(out3, a3, c3, zmax2, lin1_W, r(lin1_b), lin2_W, r(lin2_b))
    return o.reshape(N, 3, FOUT // 3)


# speedup vs baseline: 17.0197x; 1.6935x over previous
import functools

import jax
import jax.numpy as jnp
from jax import lax
from jax.experimental import pallas as pl
from jax.experimental.pallas import tpu as pltpu
from jax.experimental.pallas import tpu_sc as plsc

N = 10000
E = 320000
D = 128
H = 128
FOUT = 768
EPS = 1e-5

NC = 2
NS = 16
NW = NC * NS
EPT = E // NW
EB = 80
NP = 10240
RPT = NP // NS
ZR = 128
DEGW = 16

_MESH = plsc.VectorSubcoreMesh(core_axis_name="c", subcore_axis_name="s")

RB = 1000
NB = N // RB


NITER = EPT // EB
NBUF = 5


def _make_sc_degree():
    @functools.partial(
        pl.kernel,
        out_type=jax.ShapeDtypeStruct((NC, NP, DEGW), jnp.float32),
        mesh=_MESH,
        scratch_types=[
            pltpu.VMEM((EB,), jnp.int32),
            pltpu.VMEM((EB, DEGW), jnp.float32),
            pltpu.VMEM((ZR, DEGW), jnp.float32),
            pltpu.VMEM_SHARED((NP, DEGW), jnp.float32),
        ],
    )
    def k(dst_hbm, out_hbm, dst_v, ones_v, zbuf, acc):
        cid = lax.axis_index("c")
        sid = lax.axis_index("s")
        wid = sid * NC + cid

        def fill(i, _):
            z16 = jnp.zeros((16,), jnp.float32)
            ones_row = jnp.ones((16,), jnp.float32)
            zbuf[i, :] = z16

            @pl.when(i < EB)
            def _():
                ones_v[i, :] = ones_row
            return 0

        lax.fori_loop(0, ZR, fill, 0)
        for r in range(RPT // ZR):
            pltpu.sync_copy(zbuf, acc.at[pl.ds(sid * RPT + r * ZR, ZR)])
        plsc.subcore_barrier()

        def body(j, _):
            pltpu.sync_copy(dst_hbm.at[pl.ds(wid * EPT + j * EB, EB)], dst_v)
            pltpu.sync_copy(ones_v, acc.at[dst_v], add=True)
            return 0

        lax.fori_loop(0, EPT // EB, body, 0)
        plsc.subcore_barrier()
        pltpu.sync_copy(acc.at[pl.ds(sid * RPT, RPT)],
                        out_hbm.at[cid, pl.ds(sid * RPT, RPT)])

    return k


def _make_sc_aggregate():
    NG = 2

    @functools.partial(
        pl.kernel,
        out_type=jax.ShapeDtypeStruct((NC, NP, H), jnp.float32),
        mesh=_MESH,
        scratch_types=[
            [pltpu.VMEM((EB,), jnp.int32)] * NG,
            [pltpu.VMEM((EB,), jnp.int32)] * NG,
            [pltpu.VMEM((EB, H), jnp.float32)] * NG,
            pltpu.VMEM((ZR, H), jnp.float32),
            pltpu.VMEM_SHARED((NP, H), jnp.float32),
            [pltpu.SemaphoreType.DMA] * NG,
            [pltpu.SemaphoreType.DMA] * NG,
        ],
    )
    def k(hp_hbm, src_hbm, dst_hbm, out_hbm, srcs, dsts, gbufs, zbuf,
          acc, gsems, isems):
        cid = lax.axis_index("c")
        sid = lax.axis_index("s")
        wid = sid * NC + cid

        def fill(i, _):
            z16 = jnp.zeros((16,), jnp.float32)
            for f in range(H // 16):
                zbuf[i, pl.ds(f * 16, 16)] = z16
            return 0

        lax.fori_loop(0, ZR, fill, 0)
        for r in range(RPT // ZR):
            pltpu.sync_copy(zbuf, acc.at[pl.ds(sid * RPT + r * ZR, ZR)])
        plsc.subcore_barrier()

        def stage(slot, j):
            base = wid * EPT + j * EB
            pltpu.async_copy(dst_hbm.at[pl.ds(base, EB)],
                             dsts[slot], isems[slot])
            pltpu.sync_copy(src_hbm.at[pl.ds(base, EB)], srcs[slot])
            pltpu.async_copy(hp_hbm.at[srcs[slot]],
                             gbufs[slot], gsems[slot])

        def consume(b):
            pltpu.make_async_copy(hp_hbm.at[srcs[b]],
                                  gbufs[b], gsems[b]).wait()
            pltpu.make_async_copy(dst_hbm.at[pl.ds(0, EB)], dsts[b],
                                  isems[b]).wait()
            pltpu.sync_copy(gbufs[b], acc.at[dsts[b]], add=True)

        for b in range(NG):
            stage(b, b)

        def body(g, _):
            jj = g * NG
            for b in range(NG):
                j = jj + b
                consume(b)
                nxt = j + NG

                @pl.when(nxt < NITER)
                def _():
                    stage(b, nxt)
            return 0

        lax.fori_loop(0, NITER // NG, body, 0)
        for j in range((NITER // NG) * NG, NITER):
            consume(j % NG)
        plsc.subcore_barrier()
        pltpu.sync_copy(acc.at[pl.ds(sid * RPT, RPT)],
                        out_hbm.at[cid, pl.ds(sid * RPT, RPT)])

    return k


_sc_degree = _make_sc_degree()
_sc_aggregate = _make_sc_aggregate()


def _dinv_block(deg_ref):
    deg = deg_ref[0, :, 0:1] + deg_ref[1, :, 0:1] + 1.0
    return lax.rsqrt(deg)


_DEG_SPEC = pl.BlockSpec((NC, RB, DEGW), lambda i: (0, i, 0))
_ROW_SPEC = pl.BlockSpec((RB, H), lambda i: (i, 0))
_FULL_W = pl.BlockSpec((H, H), lambda i: (0, 0))
_VEC_SPEC = pl.BlockSpec((1, H), lambda i: (0, 0))


def _mm(a, b):
    return jnp.dot(a, b, preferred_element_type=jnp.float32,
                   precision=lax.Precision.HIGHEST)


def _tc_first(deg, x, W1):
    def body(deg_ref, x_ref, w_ref, hp_ref):
        hp_ref[...] = _mm(x_ref[...], w_ref[...]) * _dinv_block(deg_ref)

    return pl.pallas_call(
        body,
        grid=(NB,),
        in_specs=[_DEG_SPEC, _ROW_SPEC, _FULL_W],
        out_specs=_ROW_SPEC,
        out_shape=jax.ShapeDtypeStruct((N, H), jnp.float32),
    )(deg, x, W1)


def _tc_stats(S, hp, deg, b, gw, gb, gms):
    s_spec = pl.BlockSpec((NC, RB, H), lambda i: (0, i, 0))

    def body(s_ref, hp_ref, deg_ref, b_ref, gw_ref, gb_ref, gms_ref,
             out_ref, a_ref, c_ref, sum1, sum2):
        i = pl.program_id(0)

        @pl.when(i == 0)
        def _():
            sum1[...] = jnp.zeros_like(sum1)
            sum2[...] = jnp.zeros_like(sum2)

        dinv = _dinv_block(deg_ref)
        o = dinv * (s_ref[0] + s_ref[1] + hp_ref[...]) + b_ref[...]
        out_ref[...] = o
        sum1[...] += jnp.sum(o, axis=0, keepdims=True)
        sum2[...] += jnp.sum(o * o, axis=0, keepdims=True)

        @pl.when(i == NB - 1)
        def _():
            mean = sum1[...] * (1.0 / N)
            ex2 = sum2[...] * (1.0 / N)
            ms = gms_ref[...]
            var = ex2 - (2.0 * ms - ms * ms) * mean * mean
            a = gw_ref[...] / jnp.sqrt(var + EPS)
            a_ref[...] = a
            c_ref[...] = gb_ref[...] - a * ms * mean

    return pl.pallas_call(
        body,
        grid=(NB,),
        in_specs=[s_spec, _ROW_SPEC, _DEG_SPEC, _VEC_SPEC, _VEC_SPEC,
                  _VEC_SPEC, _VEC_SPEC],
        out_specs=[_ROW_SPEC, _VEC_SPEC, _VEC_SPEC],
        out_shape=[
            jax.ShapeDtypeStruct((N, H), jnp.float32),
            jax.ShapeDtypeStruct((1, H), jnp.float32),
            jax.ShapeDtypeStruct((1, H), jnp.float32),
        ],
        scratch_shapes=[pltpu.VMEM((1, H), jnp.float32),
                        pltpu.VMEM((1, H), jnp.float32)],
    )(S, hp, deg, b, gw, gb, gms)


def _tc_ad1(out1, a1, c1, deg, W2):
    def body(o_ref, a_ref, c_ref, deg_ref, w_ref, y_ref, hp_ref):
        y = jnp.maximum(a_ref[...] * o_ref[...] + c_ref[...], 0.0)
        y_ref[...] = y
        hp_ref[...] = _mm(y, w_ref[...]) * _dinv_block(deg_ref)

    return pl.pallas_call(
        body,
        grid=(NB,),
        in_specs=[_ROW_SPEC, _VEC_SPEC, _VEC_SPEC, _DEG_SPEC, _FULL_W],
        out_specs=[_ROW_SPEC, _ROW_SPEC],
        out_shape=[jax.ShapeDtypeStruct((N, H), jnp.float32),
                   jax.ShapeDtypeStruct((N, H), jnp.float32)],
    )(out1, a1, c1, deg, W2)


def _tc_ad2(out2, a2, c2, y1, deg, W3):
    def body(o_ref, a_ref, c_ref, y1_ref, deg_ref, w_ref, zmax_ref, hp_ref):
        y1 = y1_ref[...]
        y2 = jnp.maximum(a_ref[...] * o_ref[...] + c_ref[...], 0.0)
        zmax_ref[...] = jnp.maximum(y1, y2)
        hp_ref[...] = _mm(y2 + y1, w_ref[...]) * _dinv_block(deg_ref)

    return pl.pallas_call(
        body,
        grid=(NB,),
        in_specs=[_ROW_SPEC, _VEC_SPEC, _VEC_SPEC, _ROW_SPEC, _DEG_SPEC,
                  _FULL_W],
        out_specs=[_ROW_SPEC, _ROW_SPEC],
        out_shape=[jax.ShapeDtypeStruct((N, H), jnp.float32),
                   jax.ShapeDtypeStruct((N, H), jnp.float32)],
    )(out2, a2, c2, y1, deg, W3)


def _tc_final(out3, a3, c3, zmax, lin1_W, lin1_b, lin2_W, lin2_b):
    def body(o_ref, a_ref, c_ref, zm_ref, w1_ref, b1_ref, w2_ref, b2_ref,
             out_ref):
        y3 = jnp.maximum(a_ref[...] * o_ref[...] + c_ref[...], 0.0)
        z = jnp.maximum(zm_ref[...], y3)
        t = jnp.maximum(_mm(z, w1_ref[...]) + b1_ref[...], 0.0)
        out_ref[...] = _mm(t, w2_ref[...]) + b2_ref[...]

    return pl.pallas_call(
        body,
        grid=(NB,),
        in_specs=[
            _ROW_SPEC, _VEC_SPEC, _VEC_SPEC, _ROW_SPEC, _FULL_W, _VEC_SPEC,
            pl.BlockSpec((H, FOUT), lambda i: (0, 0)),
            pl.BlockSpec((1, FOUT), lambda i: (0, 0)),
        ],
        out_specs=pl.BlockSpec((RB, FOUT), lambda i: (i, 0)),
        out_shape=jax.ShapeDtypeStruct((N, FOUT), jnp.float32),
    )(out3, a3, c3, zmax, lin1_W, lin1_b, lin2_W, lin2_b)


def kernel(x, edge_index, conv_W1, conv_b1, gn_w1, gn_b1, gn_ms1,
           conv_W2, conv_b2, gn_w2, gn_b2, gn_ms2,
           conv_W3, conv_b3, gn_w3, gn_b3, gn_ms3,
           lin1_W, lin1_b, lin2_W, lin2_b):
    src1 = edge_index[0]
    dst1 = edge_index[1]
    r = lambda v: v.reshape(1, -1)

    deg = _sc_degree(dst1)
    hp1 = _tc_first(deg, x, conv_W1)
    S1 = _sc_aggregate(hp1, src1, dst1)
    out1, a1, c1 = _tc_stats(S1, hp1, deg, r(conv_b1), r(gn_w1), r(gn_b1),
                             r(gn_ms1))
    y1, hp2 = _tc_ad1(out1, a1, c1, deg, conv_W2)
    S2 = _sc_aggregate(hp2, src1, dst1)
    out2, a2, c2 = _tc_stats(S2, hp2, deg, r(conv_b2), r(gn_w2), r(gn_b2),
                             r(gn_ms2))
    zmax2, hp3 = _tc_ad2(out2, a2, c2, y1, deg, conv_W3)
    S3 = _sc_aggregate(hp3, src1, dst1)
    out3, a3, c3 = _tc_stats(S3, hp3, deg, r(conv_b3), r(gn_w3), r(gn_b3),
                             r(gn_ms3))
    o = _tc_final(out3, a3, c3, zmax2, lin1_W, r(lin1_b), lin2_W, r(lin2_b))
    return o.reshape(N, 3, FOUT // 3)


# speedup vs baseline: 21.4286x; 1.2590x over previous
import functools

import jax
import jax.numpy as jnp
from jax import lax
from jax.experimental import pallas as pl
from jax.experimental.pallas import tpu as pltpu
from jax.experimental.pallas import tpu_sc as plsc

N = 10000
E = 320000
D = 128
H = 128
FOUT = 768
EPS = 1e-5

NC = 2
NS = 16
NW = NC * NS
EPT = E // NW
EB = 80
NP = 10240
RPT = NP // NS
ZR = 128
DEGW = 16

_MESH = plsc.VectorSubcoreMesh(core_axis_name="c", subcore_axis_name="s")

RB = 1000
NB = N // RB


NITER = EPT // EB
NBUF = 5


def _make_sc_degree():
    @functools.partial(
        pl.kernel,
        out_type=jax.ShapeDtypeStruct((NC, NP, DEGW), jnp.float32),
        mesh=_MESH,
        scratch_types=[
            [pltpu.VMEM((EB,), jnp.int32)] * 4,
            pltpu.VMEM((EB, DEGW), jnp.float32),
            pltpu.VMEM((ZR, DEGW), jnp.float32),
            pltpu.VMEM_SHARED((NP, DEGW), jnp.float32),
            [pltpu.SemaphoreType.DMA] * 4,
        ],
    )
    def k(dst_hbm, out_hbm, dsts, ones_v, zbuf, acc, dsems):
        cid = lax.axis_index("c")
        sid = lax.axis_index("s")
        wid = sid * NC + cid

        def stage(si, j):
            pltpu.async_copy(dst_hbm.at[pl.ds(wid * EPT + j * EB, EB)],
                             dsts[si], dsems[si])

        for j in range(4):
            stage(j, j)

        def fill(i, _):
            z16 = jnp.zeros((16,), jnp.float32)
            ones_row = jnp.ones((16,), jnp.float32)
            zbuf[i, :] = z16

            @pl.when(i < EB)
            def _():
                ones_v[i, :] = ones_row
            return 0

        lax.fori_loop(0, ZR, fill, 0)
        for r in range(RPT // ZR):
            pltpu.sync_copy(zbuf, acc.at[pl.ds(sid * RPT + r * ZR, ZR)])
        plsc.subcore_barrier()

        def body(g, _):
            jj = g * 4
            for u in range(4):
                j = jj + u
                pltpu.make_async_copy(dst_hbm.at[pl.ds(0, EB)], dsts[u],
                                      dsems[u]).wait()
                pltpu.sync_copy(ones_v, acc.at[dsts[u]], add=True)
                ni = j + 4

                @pl.when(ni < NITER)
                def _():
                    stage(u, ni)
            return 0

        lax.fori_loop(0, NITER // 4, body, 0)
        for r in range(NITER - (NITER // 4) * 4):
            pltpu.make_async_copy(dst_hbm.at[pl.ds(0, EB)], dsts[r],
                                  dsems[r]).wait()
            pltpu.sync_copy(ones_v, acc.at[dsts[r]], add=True)
        plsc.subcore_barrier()
        pltpu.sync_copy(acc.at[pl.ds(sid * RPT, RPT)],
                        out_hbm.at[cid, pl.ds(sid * RPT, RPT)])

    return k


def _make_sc_aggregate():
    NG = 2
    NI = 4

    @functools.partial(
        pl.kernel,
        out_type=jax.ShapeDtypeStruct((NC, NP, H), jnp.float32),
        mesh=_MESH,
        scratch_types=[
            [pltpu.VMEM((EB,), jnp.int32)] * NI,
            [pltpu.VMEM((EB,), jnp.int32)] * NI,
            [pltpu.VMEM((EB, H), jnp.float32)] * NG,
            pltpu.VMEM((ZR, H), jnp.float32),
            pltpu.VMEM_SHARED((NP, H), jnp.float32),
            [pltpu.SemaphoreType.DMA] * NG,
            [pltpu.SemaphoreType.DMA] * NI,
            [pltpu.SemaphoreType.DMA] * NI,
        ],
    )
    def k(hp_hbm, src_hbm, dst_hbm, out_hbm, srcs, dsts, gbufs, zbuf,
          acc, gsems, ssems, dsems):
        cid = lax.axis_index("c")
        sid = lax.axis_index("s")
        wid = sid * NC + cid

        def stage_idx(si, j):
            base = wid * EPT + j * EB
            pltpu.async_copy(src_hbm.at[pl.ds(base, EB)], srcs[si], ssems[si])
            pltpu.async_copy(dst_hbm.at[pl.ds(base, EB)], dsts[si], dsems[si])

        def stage_gather(b, si):
            pltpu.make_async_copy(src_hbm.at[pl.ds(0, EB)], srcs[si],
                                  ssems[si]).wait()
            pltpu.async_copy(hp_hbm.at[srcs[si]], gbufs[b], gsems[b])

        def consume(b, si):
            pltpu.make_async_copy(hp_hbm.at[srcs[si]], gbufs[b],
                                  gsems[b]).wait()
            pltpu.make_async_copy(dst_hbm.at[pl.ds(0, EB)], dsts[si],
                                  dsems[si]).wait()
            pltpu.sync_copy(gbufs[b], acc.at[dsts[si]], add=True)

        for j in range(NI):
            stage_idx(j, j)

        def fill(i, _):
            z16 = jnp.zeros((16,), jnp.float32)
            for f in range(H // 16):
                zbuf[i, pl.ds(f * 16, 16)] = z16
            return 0

        lax.fori_loop(0, ZR, fill, 0)
        for r in range(RPT // ZR):
            pltpu.sync_copy(zbuf, acc.at[pl.ds(sid * RPT + r * ZR, ZR)])
        plsc.subcore_barrier()

        for j in range(NG):
            stage_gather(j % NG, j % NI)

        def body(g, _):
            jj = g * NI
            for u in range(NI):
                j = jj + u
                b = u % NG
                si = u
                consume(b, si)
                ni = j + NI

                @pl.when(ni < NITER)
                def _():
                    stage_idx(si, ni)
                ng = j + NG

                @pl.when(ng < NITER)
                def _():
                    stage_gather(b, (u + NG) % NI)
            return 0

        lax.fori_loop(0, NITER // NI, body, 0)
        rem = NITER - (NITER // NI) * NI
        for r in range(rem):
            j = (NITER // NI) * NI + r
            b = r % NG
            si = r
            consume(b, si)
            ng = j + NG
            if ng < NITER:
                stage_gather(b, (r + NG) % NI)
        plsc.subcore_barrier()
        pltpu.sync_copy(acc.at[pl.ds(sid * RPT, RPT)],
                        out_hbm.at[cid, pl.ds(sid * RPT, RPT)])

    return k


_sc_degree = _make_sc_degree()
_sc_aggregate = _make_sc_aggregate()


def _dinv_block(deg_ref):
    deg = deg_ref[0, :, 0:1] + deg_ref[1, :, 0:1] + 1.0
    return lax.rsqrt(deg)


_DEG_SPEC = pl.BlockSpec((NC, RB, DEGW), lambda i: (0, i, 0))
_ROW_SPEC = pl.BlockSpec((RB, H), lambda i: (i, 0))
_FULL_W = pl.BlockSpec((H, H), lambda i: (0, 0))
_VEC_SPEC = pl.BlockSpec((1, H), lambda i: (0, 0))


def _mm(a, b):
    return jnp.dot(a, b, preferred_element_type=jnp.float32,
                   precision=lax.Precision.HIGHEST)


def _tc_first(deg, x, W1):
    def body(deg_ref, x_ref, w_ref, hp_ref):
        hp_ref[...] = _mm(x_ref[...], w_ref[...]) * _dinv_block(deg_ref)

    return pl.pallas_call(
        body,
        grid=(NB,),
        in_specs=[_DEG_SPEC, _ROW_SPEC, _FULL_W],
        out_specs=_ROW_SPEC,
        out_shape=jax.ShapeDtypeStruct((N, H), jnp.float32),
    )(deg, x, W1)


def _tc_stats(S, hp, deg, b, gw, gb, gms):
    s_spec = pl.BlockSpec((NC, RB, H), lambda i: (0, i, 0))

    def body(s_ref, hp_ref, deg_ref, b_ref, gw_ref, gb_ref, gms_ref,
             out_ref, a_ref, c_ref, sum1, sum2):
        i = pl.program_id(0)

        @pl.when(i == 0)
        def _():
            sum1[...] = jnp.zeros_like(sum1)
            sum2[...] = jnp.zeros_like(sum2)

        dinv = _dinv_block(deg_ref)
        o = dinv * (s_ref[0] + s_ref[1] + hp_ref[...]) + b_ref[...]
        out_ref[...] = o
        sum1[...] += jnp.sum(o, axis=0, keepdims=True)
        sum2[...] += jnp.sum(o * o, axis=0, keepdims=True)

        @pl.when(i == NB - 1)
        def _():
            mean = sum1[...] * (1.0 / N)
            ex2 = sum2[...] * (1.0 / N)
            ms = gms_ref[...]
            var = ex2 - (2.0 * ms - ms * ms) * mean * mean
            a = gw_ref[...] / jnp.sqrt(var + EPS)
            a_ref[...] = a
            c_ref[...] = gb_ref[...] - a * ms * mean

    return pl.pallas_call(
        body,
        grid=(NB,),
        in_specs=[s_spec, _ROW_SPEC, _DEG_SPEC, _VEC_SPEC, _VEC_SPEC,
                  _VEC_SPEC, _VEC_SPEC],
        out_specs=[_ROW_SPEC, _VEC_SPEC, _VEC_SPEC],
        out_shape=[
            jax.ShapeDtypeStruct((N, H), jnp.float32),
            jax.ShapeDtypeStruct((1, H), jnp.float32),
            jax.ShapeDtypeStruct((1, H), jnp.float32),
        ],
        scratch_shapes=[pltpu.VMEM((1, H), jnp.float32),
                        pltpu.VMEM((1, H), jnp.float32)],
    )(S, hp, deg, b, gw, gb, gms)


def _tc_ad1(out1, a1, c1, deg, W2):
    def body(o_ref, a_ref, c_ref, deg_ref, w_ref, y_ref, hp_ref):
        y = jnp.maximum(a_ref[...] * o_ref[...] + c_ref[...], 0.0)
        y_ref[...] = y
        hp_ref[...] = _mm(y, w_ref[...]) * _dinv_block(deg_ref)

    return pl.pallas_call(
        body,
        grid=(NB,),
        in_specs=[_ROW_SPEC, _VEC_SPEC, _VEC_SPEC, _DEG_SPEC, _FULL_W],
        out_specs=[_ROW_SPEC, _ROW_SPEC],
        out_shape=[jax.ShapeDtypeStruct((N, H), jnp.float32),
                   jax.ShapeDtypeStruct((N, H), jnp.float32)],
    )(out1, a1, c1, deg, W2)


def _tc_ad2(out2, a2, c2, y1, deg, W3):
    def body(o_ref, a_ref, c_ref, y1_ref, deg_ref, w_ref, zmax_ref, hp_ref):
        y1 = y1_ref[...]
        y2 = jnp.maximum(a_ref[...] * o_ref[...] + c_ref[...], 0.0)
        zmax_ref[...] = jnp.maximum(y1, y2)
        hp_ref[...] = _mm(y2 + y1, w_ref[...]) * _dinv_block(deg_ref)

    return pl.pallas_call(
        body,
        grid=(NB,),
        in_specs=[_ROW_SPEC, _VEC_SPEC, _VEC_SPEC, _ROW_SPEC, _DEG_SPEC,
                  _FULL_W],
        out_specs=[_ROW_SPEC, _ROW_SPEC],
        out_shape=[jax.ShapeDtypeStruct((N, H), jnp.float32),
                   jax.ShapeDtypeStruct((N, H), jnp.float32)],
    )(out2, a2, c2, y1, deg, W3)


def _tc_final(out3, a3, c3, zmax, lin1_W, lin1_b, lin2_W, lin2_b):
    def body(o_ref, a_ref, c_ref, zm_ref, w1_ref, b1_ref, w2_ref, b2_ref,
             out_ref):
        y3 = jnp.maximum(a_ref[...] * o_ref[...] + c_ref[...], 0.0)
        z = jnp.maximum(zm_ref[...], y3)
        t = jnp.maximum(_mm(z, w1_ref[...]) + b1_ref[...], 0.0)
        for p in range(3):
            out_ref[:, p, :] = (_mm(t, w2_ref[:, p, :])
                                + b2_ref[:, p, :][0])

    return pl.pallas_call(
        body,
        grid=(NB,),
        in_specs=[
            _ROW_SPEC, _VEC_SPEC, _VEC_SPEC, _ROW_SPEC, _FULL_W, _VEC_SPEC,
            pl.BlockSpec((H, 3, FOUT // 3), lambda i: (0, 0, 0)),
            pl.BlockSpec((1, 3, FOUT // 3), lambda i: (0, 0, 0)),
        ],
        out_specs=pl.BlockSpec((RB, 3, FOUT // 3), lambda i: (i, 0, 0)),
        out_shape=jax.ShapeDtypeStruct((N, 3, FOUT // 3), jnp.float32),
    )(out3, a3, c3, zmax, lin1_W, lin1_b,
      lin2_W.reshape(H, 3, FOUT // 3), lin2_b.reshape(1, 3, FOUT // 3))


def kernel(x, edge_index, conv_W1, conv_b1, gn_w1, gn_b1, gn_ms1,
           conv_W2, conv_b2, gn_w2, gn_b2, gn_ms2,
           conv_W3, conv_b3, gn_w3, gn_b3, gn_ms3,
           lin1_W, lin1_b, lin2_W, lin2_b):
    src1 = edge_index[0]
    dst1 = edge_index[1]
    r = lambda v: v.reshape(1, -1)

    deg = _sc_degree(dst1)
    hp1 = _tc_first(deg, x, conv_W1)
    S1 = _sc_aggregate(hp1, src1, dst1)
    out1, a1, c1 = _tc_stats(S1, hp1, deg, r(conv_b1), r(gn_w1), r(gn_b1),
                             r(gn_ms1))
    y1, hp2 = _tc_ad1(out1, a1, c1, deg, conv_W2)
    S2 = _sc_aggregate(hp2, src1, dst1)
    out2, a2, c2 = _tc_stats(S2, hp2, deg, r(conv_b2), r(gn_w2), r(gn_b2),
                             r(gn_ms2))
    zmax2, hp3 = _tc_ad2(out2, a2, c2, y1, deg, conv_W3)
    S3 = _sc_aggregate(hp3, src1, dst1)
    out3, a3, c3 = _tc_stats(S3, hp3, deg, r(conv_b3), r(gn_w3), r(gn_b3),
                             r(gn_ms3))
    return _tc_final(out3, a3, c3, zmax2, lin1_W, r(lin1_b), lin2_W, lin2_b)
